# baseline (speedup 1.0000x reference)
import functools
import numpy as np
import jax
from jax import lax
import jax.numpy as jnp
from jax.experimental import pallas as pl
from jax.experimental.pallas import tpu as pltpu
from jax.experimental.pallas import tpu_sc as plsc

_SLOW_FRAMES = 8


def _slow_indices(t):
    return tuple(int(v) for v in np.linspace(0, t - 1, _SLOW_FRAMES).astype(np.int32))


def kernel(frames):
    b, t, c, h, w = frames.shape
    n_slow = _SLOW_FRAMES
    assert _slow_indices(t) == tuple((j * (t - 1)) // (n_slow - 1) for j in range(n_slow))
    mesh = plsc.VectorSubcoreMesh(core_axis_name="c", subcore_axis_name="s")
    n_workers = 32
    chunks = b * n_slow
    per_w = chunks // n_workers
    hh = h // 2
    ppc = c * 2
    n_pieces = per_w * ppc

    @functools.partial(
        pl.kernel,
        mesh=mesh,
        out_type=jax.ShapeDtypeStruct((b, n_slow, c, h, w), frames.dtype),
        scratch_types=[
            pltpu.VMEM((hh, w), frames.dtype),
            pltpu.VMEM((hh, w), frames.dtype),
            pltpu.SemaphoreType.DMA,
            pltpu.SemaphoreType.DMA,
            pltpu.SemaphoreType.DMA,
            pltpu.SemaphoreType.DMA,
        ],
    )
    def sc_gather(frames_hbm, slow_hbm, stage0, stage1, in0, in1, out0, out1):
        cid = lax.axis_index("c")
        sid = lax.axis_index("s")
        wid = sid * 2 + cid
        stages = (stage0, stage1)
        in_sems = (in0, in1)
        out_sems = (out0, out1)

        def coords(p):
            r = wid * per_w + p // ppc
            q = p % ppc
            ci, half = q // 2, q % 2
            bi = r // n_slow
            j = r % n_slow
            ti = (j * (t - 1)) // (n_slow - 1)
            return bi, j, ti, ci, half * hh

        def fetch(p):
            k = p % 2
            bi, _, ti, ci, row0 = coords(p)
            cp = pltpu.make_async_copy(
                frames_hbm.at[bi, ti, ci, pl.ds(row0, hh)], stages[k], in_sems[k]
            )
            cp.start()
            return cp

        def put(p):
            k = p % 2
            bi, j, _, ci, row0 = coords(p)
            cp = pltpu.make_async_copy(
                stages[k], slow_hbm.at[bi, j, ci, pl.ds(row0, hh)], out_sems[k]
            )
            cp.start()
            return cp

        ins = {0: fetch(0)}
        outs = {}
        for p in range(n_pieces):
            ins[p].wait()
            outs[p] = put(p)
            if p + 1 < n_pieces:
                if p >= 1:
                    outs[p - 1].wait()
                ins[p + 1] = fetch(p + 1)
        outs[n_pieces - 2].wait()
        outs[n_pieces - 1].wait()

    slow = sc_gather(frames)
    return (slow, frames)


# speedup vs baseline: 1.1529x; 1.1529x over previous
import functools
import numpy as np
import jax
import jax.numpy as jnp
from jax.experimental import pallas as pl
from jax.experimental.pallas import tpu as pltpu

_SLOW_FRAMES = 8
_K = 8


def _slow_indices(t):
    return tuple(int(v) for v in np.linspace(0, t - 1, _SLOW_FRAMES).astype(np.int32))


def _gather_ring_kernel(coords, frames_ref, slow_ref, *scratch):
    n = len(coords)
    bufs = scratch[:_K]
    in_sems = scratch[_K : 2 * _K]
    out_sems = scratch[2 * _K : 3 * _K]
    lead = _K // 2
    ins, outs = {}, {}
    for step in range(n + lead):
        if step < n:
            k = step % _K
            bi, j, ti = coords[step]
            if step >= _K:
                outs[step - _K].wait()
            cp = pltpu.make_async_copy(frames_ref.at[bi, ti], bufs[k], in_sems[k])
            cp.start()
            ins[step] = cp
        r = step - lead
        if r >= 0:
            k = r % _K
            bi, j, ti = coords[r]
            ins[r].wait()
            cp = pltpu.make_async_copy(bufs[k], slow_ref.at[bi, j], out_sems[k])
            cp.start()
            outs[r] = cp
    for r in range(max(0, n - _K), n):
        outs[r].wait()


def kernel(frames):
    b, t, c, h, w = frames.shape
    n_slow = _SLOW_FRAMES
    idx = _slow_indices(t)
    coords = tuple((bi, j, idx[j]) for bi in range(b) for j in range(n_slow))

    slow = pl.pallas_call(
        functools.partial(_gather_ring_kernel, coords),
        out_shape=jax.ShapeDtypeStruct((b, n_slow, c, h, w), frames.dtype),
        in_specs=[pl.BlockSpec(memory_space=pl.ANY)],
        out_specs=pl.BlockSpec(memory_space=pl.ANY),
        scratch_shapes=(
            [pltpu.VMEM((c, h, w), frames.dtype)] * _K
            + [pltpu.SemaphoreType.DMA] * (2 * _K)
        ),
    )(frames)
    return (slow, frames)
